# baseline (speedup 1.0000x reference)
import jax
import jax.numpy as jnp
from jax.experimental import pallas as pl
from jax.experimental.pallas import tpu as pltpu

_ALPHA = 1.702
_LIMIT = 7.0


def _moe_body(ri_ref, rw_ref, x_ref, wgu_ref, bgu_ref, wd_ref, bd_ref, out_ref):
    e = pl.program_id(0)

    @pl.when(e == 0)
    def _init():
        out_ref[...] = jnp.zeros_like(out_ref)

    x = x_ref[...].astype(jnp.bfloat16)
    wgu = wgu_ref[0].astype(jnp.bfloat16)
    gu = jnp.dot(x, wgu, preferred_element_type=jnp.float32) + bgu_ref[0, 0]
    gu3 = gu.reshape(gu.shape[0], gu.shape[1] // 2, 2)
    gate = gu3[:, :, 0]
    up = gu3[:, :, 1]
    gate = jnp.minimum(gate, _LIMIT)
    up = jnp.clip(up, -_LIMIT, _LIMIT)
    glu = gate * jax.nn.sigmoid(gate * _ALPHA)
    gated = (up + 1.0) * glu
    out = jnp.dot(gated.astype(jnp.bfloat16), wd_ref[0].astype(jnp.bfloat16),
                  preferred_element_type=jnp.float32) + bd_ref[0, 0]
    w = jnp.sum(rw_ref[...] * (ri_ref[...] == e).astype(jnp.float32), axis=1,
                keepdims=True)
    out_ref[...] += out * w


def kernel(hidden_states, router_indices, routing_weights, gate_up_proj,
           gate_up_proj_bias, down_proj, down_proj_bias):
    T, H = hidden_states.shape
    E, _, I2 = gate_up_proj.shape
    I = I2 // 2

    bgu3 = gate_up_proj_bias.reshape(E, 1, I2)
    bd3 = down_proj_bias.reshape(E, 1, H)

    grid = (E,)
    out = pl.pallas_call(
        _moe_body,
        grid=grid,
        in_specs=[
            pl.BlockSpec((T, router_indices.shape[1]), lambda e: (0, 0)),
            pl.BlockSpec((T, routing_weights.shape[1]), lambda e: (0, 0)),
            pl.BlockSpec((T, H), lambda e: (0, 0)),
            pl.BlockSpec((1, H, I2), lambda e: (e, 0, 0)),
            pl.BlockSpec((1, 1, I2), lambda e: (e, 0, 0)),
            pl.BlockSpec((1, I, H), lambda e: (e, 0, 0)),
            pl.BlockSpec((1, 1, H), lambda e: (e, 0, 0)),
        ],
        out_specs=pl.BlockSpec((T, H), lambda e: (0, 0)),
        out_shape=jax.ShapeDtypeStruct((T, H), hidden_states.dtype),
        compiler_params=pltpu.CompilerParams(
            dimension_semantics=("arbitrary",),
        ),
    )(router_indices, routing_weights, hidden_states, gate_up_proj, bgu3,
      down_proj, bd3)
    return out


# speedup vs baseline: 1.1493x; 1.1493x over previous
import jax
import jax.numpy as jnp
from jax.experimental import pallas as pl
from jax.experimental.pallas import tpu as pltpu

_ALPHA = 1.702
_LIMIT = 7.0


def _moe_body(ri_ref, rw_ref, x_ref, wgu_ref, bgu_ref, wd_ref, bd_ref, out_ref):
    e = pl.program_id(0)

    @pl.when(e == 0)
    def _init():
        out_ref[...] = jnp.zeros_like(out_ref)

    x = x_ref[...]
    gu = jnp.dot(x, wgu_ref[0], preferred_element_type=jnp.float32) + bgu_ref[0, 0]
    gu3 = gu.reshape(gu.shape[0], gu.shape[1] // 2, 2)
    gate = gu3[:, :, 0]
    up = gu3[:, :, 1]
    gate = jnp.minimum(gate, _LIMIT)
    up = jnp.clip(up, -_LIMIT, _LIMIT)
    glu = gate * jax.nn.sigmoid(gate * _ALPHA)
    gated = (up + 1.0) * glu
    out = jnp.dot(gated, wd_ref[0], preferred_element_type=jnp.float32) + bd_ref[0, 0]
    w = jnp.sum(rw_ref[...] * (ri_ref[...] == e).astype(jnp.float32), axis=1,
                keepdims=True)
    out_ref[...] += out * w


def kernel(hidden_states, router_indices, routing_weights, gate_up_proj,
           gate_up_proj_bias, down_proj, down_proj_bias):
    T, H = hidden_states.shape
    E, _, I2 = gate_up_proj.shape
    I = I2 // 2

    bgu3 = gate_up_proj_bias.reshape(E, 1, I2)
    bd3 = down_proj_bias.reshape(E, 1, H)

    grid = (E,)
    out = pl.pallas_call(
        _moe_body,
        grid=grid,
        in_specs=[
            pl.BlockSpec((T, router_indices.shape[1]), lambda e: (0, 0)),
            pl.BlockSpec((T, routing_weights.shape[1]), lambda e: (0, 0)),
            pl.BlockSpec((T, H), lambda e: (0, 0)),
            pl.BlockSpec((1, H, I2), lambda e: (e, 0, 0)),
            pl.BlockSpec((1, 1, I2), lambda e: (e, 0, 0)),
            pl.BlockSpec((1, I, H), lambda e: (e, 0, 0)),
            pl.BlockSpec((1, 1, H), lambda e: (e, 0, 0)),
        ],
        out_specs=pl.BlockSpec((T, H), lambda e: (0, 0)),
        out_shape=jax.ShapeDtypeStruct((T, H), hidden_states.dtype),
        compiler_params=pltpu.CompilerParams(
            dimension_semantics=("arbitrary",),
        ),
    )(router_indices, routing_weights, hidden_states, gate_up_proj, bgu3,
      down_proj, bd3)
    return out


# speedup vs baseline: 3.5724x; 3.1084x over previous
import jax
import jax.numpy as jnp
from jax.experimental import pallas as pl
from jax.experimental.pallas import tpu as pltpu

_ALPHA = 1.702
_LIMIT = 7.0


def _moe_body(ri_ref, rw_ref, x_ref, wgu_ref, bgu_ref, wd_ref, bd_ref,
              out_ref, sel_even_ref, sel_odd_ref):
    e = pl.program_id(0)
    I2 = wgu_ref.shape[2]
    I = I2 // 2

    @pl.when(e == 0)
    def _init():
        out_ref[...] = jnp.zeros_like(out_ref)
        row = jax.lax.broadcasted_iota(jnp.int32, (I2, I), 0)
        col = jax.lax.broadcasted_iota(jnp.int32, (I2, I), 1)
        sel_even_ref[...] = (row == 2 * col).astype(jnp.float32)
        sel_odd_ref[...] = (row == 2 * col + 1).astype(jnp.float32)

    x = x_ref[...]
    gu = jnp.dot(x, wgu_ref[0], preferred_element_type=jnp.float32) + bgu_ref[0, 0]
    gate_full = jnp.minimum(gu, _LIMIT)
    up_full = jnp.clip(gu, -_LIMIT, _LIMIT)
    glu_full = gate_full * jax.nn.sigmoid(gate_full * _ALPHA)
    glu = jnp.dot(glu_full, sel_even_ref[...], preferred_element_type=jnp.float32)
    up = jnp.dot(up_full, sel_odd_ref[...], preferred_element_type=jnp.float32)
    gated = (up + 1.0) * glu
    out = jnp.dot(gated, wd_ref[0], preferred_element_type=jnp.float32) + bd_ref[0, 0]
    w = jnp.sum(rw_ref[...] * (ri_ref[...] == e).astype(jnp.float32), axis=1,
                keepdims=True)
    out_ref[...] += out * w


def kernel(hidden_states, router_indices, routing_weights, gate_up_proj,
           gate_up_proj_bias, down_proj, down_proj_bias):
    T, H = hidden_states.shape
    E, _, I2 = gate_up_proj.shape
    I = I2 // 2

    bgu3 = gate_up_proj_bias.reshape(E, 1, I2)
    bd3 = down_proj_bias.reshape(E, 1, H)

    out = pl.pallas_call(
        _moe_body,
        grid=(E,),
        in_specs=[
            pl.BlockSpec((T, router_indices.shape[1]), lambda e: (0, 0)),
            pl.BlockSpec((T, routing_weights.shape[1]), lambda e: (0, 0)),
            pl.BlockSpec((T, H), lambda e: (0, 0)),
            pl.BlockSpec((1, H, I2), lambda e: (e, 0, 0)),
            pl.BlockSpec((1, 1, I2), lambda e: (e, 0, 0)),
            pl.BlockSpec((1, I, H), lambda e: (e, 0, 0)),
            pl.BlockSpec((1, 1, H), lambda e: (e, 0, 0)),
        ],
        out_specs=pl.BlockSpec((T, H), lambda e: (0, 0)),
        out_shape=jax.ShapeDtypeStruct((T, H), hidden_states.dtype),
        scratch_shapes=[
            pltpu.VMEM((I2, I), jnp.float32),
            pltpu.VMEM((I2, I), jnp.float32),
        ],
        compiler_params=pltpu.CompilerParams(
            dimension_semantics=("arbitrary",),
        ),
    )(router_indices, routing_weights, hidden_states, gate_up_proj, bgu3,
      down_proj, bd3)
    return out
